# baseline (speedup 1.0000x reference)
import functools

import jax
import jax.numpy as jnp
from jax import lax
from jax.experimental import pallas as pl
from jax.experimental.pallas import tpu as pltpu
from jax.experimental.pallas import tpu_sc as plsc

MAX_PATH = 5
MAX_VISITS = 1
_NEG_INF = float("-inf")


def _emb_body(L, D, tok_per_w, ids_hbm, emb_hbm, out_hbm, idx_v, rows0_v,
              rows1_v, acc_v, sem0, sem1):
    nc = 2
    wid = lax.axis_index("s") * nc + lax.axis_index("c")
    pltpu.sync_copy(ids_hbm.at[wid], idx_v)

    inv = jnp.float32(1.0) / jnp.float32(L)

    def _fold_tree(vals):
        while len(vals) > 1:
            half = (len(vals) + 1) // 2
            vals = [vals[i] + vals[i + half] if i + half < len(vals)
                    else vals[i] for i in range(half)]
        return vals[0]

    def _accumulate(rows_v, t):
        def col_body(c4, _):
            for u in range(4):
                c = 4 * c4 + u
                acc = None
                for j in range(0, L, 8):
                    n = min(8, L - j)
                    vals = [rows_v[j + r, pl.ds(16 * c, 16)]
                            for r in range(n)]
                    tj = _fold_tree(vals)
                    acc = tj if acc is None else acc + tj
                acc_v[pl.ds(16 * c, 16)] = acc * inv
            return 0

        lax.fori_loop(0, D // 64, col_body, 0)
        pltpu.sync_copy(acc_v, out_hbm.at[wid * tok_per_w + t])

    pltpu.async_copy(emb_hbm.at[idx_v.at[0]], rows0_v, sem0)

    def pair_body(i, _):
        t0 = 2 * i
        t1 = t0 + 1
        pltpu.async_copy(emb_hbm.at[idx_v.at[t1]], rows1_v, sem1)
        pltpu.make_async_copy(emb_hbm.at[idx_v.at[t0]], rows0_v, sem0).wait()
        _accumulate(rows0_v, t0)

        @pl.when(t0 + 2 < tok_per_w)
        def _next():
            pltpu.async_copy(emb_hbm.at[idx_v.at[t0 + 2]], rows0_v, sem0)

        pltpu.make_async_copy(emb_hbm.at[idx_v.at[t1]], rows1_v, sem1).wait()
        _accumulate(rows1_v, t1)
        return 0

    lax.fori_loop(0, tok_per_w // 2, pair_body, 0)


def _embed_sc(input_ids, emb):
    B, L = input_ids.shape
    V, D = emb.shape
    nw = 32
    tok_per_w = B // nw
    l_pad = (L + 7) // 8 * 8
    ids = jnp.pad(input_ids.astype(jnp.int32), ((0, 0), (0, l_pad - L)))
    ids = ids.reshape(nw, tok_per_w, l_pad)
    mesh = plsc.VectorSubcoreMesh(core_axis_name="c", subcore_axis_name="s")
    body = functools.partial(_emb_body, L, D, tok_per_w)
    return pl.kernel(
        body,
        out_type=jax.ShapeDtypeStruct((B, D), jnp.float32),
        mesh=mesh,
        scratch_types=[
            pltpu.VMEM((tok_per_w, l_pad), jnp.int32),
            pltpu.VMEM((l_pad, D), jnp.float32),
            pltpu.VMEM((l_pad, D), jnp.float32),
            pltpu.VMEM((D,), jnp.float32),
            pltpu.SemaphoreType.DMA,
            pltpu.SemaphoreType.DMA,
        ],
    )(ids, emb)


def _moe_body(E, x_in, Wr1, br1, Wr2, br2, We1, be1, We2, be2, tags, Wc, bc,
              one, cls_out, ent_out,
              x_s, out_s, visits_s, eidx_s, active_s, ent_s):
    s = pl.program_id(0)
    j = pl.program_id(1)
    B = x_s.shape[0]
    W = visits_s.shape[1]

    @pl.when((s == 0) & (j == 0))
    def _init():
        x_s[...] = x_in[...]
        visits_s[...] = jnp.zeros_like(visits_s)
        active_s[...] = jnp.ones_like(active_s)
        ent_s[0, 0] = 0.0

    @pl.when(j == 0)
    def _router():
        x = x_s[...]
        h = jnp.maximum(
            jnp.dot(x, Wr1[...], preferred_element_type=jnp.float32)
            + br1[...], 0.0)
        logits = (jnp.dot(h, Wr2[...], preferred_element_type=jnp.float32)
                  + br2[...])
        blocked = visits_s[...] >= MAX_VISITS
        logits = jnp.where(blocked, _NEG_INF, logits)
        m = jnp.max(logits, axis=1, keepdims=True)
        ex = jnp.exp(logits - m)
        probs = ex / jnp.sum(ex, axis=1, keepdims=True)
        ent = -jnp.sum(probs * jnp.log(probs + 1e-9), axis=1)
        af = active_s[...][:, 0]
        cnt = jnp.sum(af)
        step_ent = jnp.sum(ent * af) / jnp.maximum(cnt, 1.0)
        ent_s[0, 0] = ent_s[0, 0] + jnp.where(cnt > 0.0, step_ent, 0.0)
        sel = jnp.argmax(logits, axis=1).astype(jnp.int32)
        eidx = jnp.minimum(sel, E - 1)
        upd = (af > 0.0) & (sel != E)
        updf = upd.astype(jnp.float32)
        eidx_s[...] = eidx[:, None]
        active_s[...] = updf[:, None]
        col = lax.broadcasted_iota(jnp.int32, (B, W), 1)
        oh = ((col == eidx[:, None]) & upd[:, None]).astype(jnp.float32)
        visits_s[...] = visits_s[...] + oh
        out_s[...] = jnp.zeros_like(out_s)

    @pl.when(j > 0)
    def _expert():
        e = j - 1
        msk = (eidx_s[...] == e) & (active_s[...] > 0.0)

        @pl.when(jnp.any(msk))
        def _apply():
            D = x_s.shape[1]
            HE = We1.shape[2]
            s1 = one[0, 0]
            HH = HE // 2
            acc2 = None
            for half in range(2):
                hsl = slice(half * HH, (half + 1) * HH)
                acc1 = None
                for j in range(0, D, 128):
                    p = jnp.dot(x_s[:, j:j + 128], We1[0][j:j + 128, hsl],
                                preferred_element_type=jnp.float32) * s1
                    acc1 = p if acc1 is None else acc1 + p
                h1 = jnp.maximum(acc1 + be1[0][:, hsl], 0.0)
                for k in range(0, HH, 256):
                    ka = half * HH + k
                    pr = jnp.dot(h1[:, k:k + 128], We2[0][ka:ka + 128, :],
                                 preferred_element_type=jnp.float32) * s1
                    if k + 128 < HH:
                        pb = jnp.dot(h1[:, k + 128:k + 256],
                                     We2[0][ka + 128:ka + 256, :],
                                     preferred_element_type=jnp.float32) * s1
                        pr = pr + pb
                    acc2 = pr if acc2 is None else acc2 + pr
            o = acc2 + be2[0] + tags[0]
            out_s[...] = jnp.where(msk, o, out_s[...])

        @pl.when(j == E)
        def _finalize():
            xf = jnp.where(active_s[...] > 0.0, out_s[...], x_s[...])
            x_s[...] = xf

            @pl.when(s == MAX_PATH - 1)
            def _emit():
                cls_out[...] = (
                    jnp.dot(xf, Wc[...], preferred_element_type=jnp.float32)
                    + bc[...])
                ent_out[...] = jnp.full((1, 1), ent_s[0, 0], jnp.float32)


def _moe_tc(x, We1, be1, We2, be2, tags, Wr1, br1, Wr2, br2, Wc, bc):
    B, D = x.shape
    E, _, HE = We1.shape
    HR = Wr1.shape[1]
    C = Wc.shape[1]
    W = 16
    Wr2p = jnp.pad(Wr2, ((0, 0), (0, W - (E + 1))))
    br2p = jnp.pad(br2, (0, W - (E + 1)), constant_values=_NEG_INF)

    def expert_map(s, j):
        return (jnp.maximum(j - 1, 0), 0, 0)

    def expert_map2(s, j):
        return (jnp.maximum(j - 1, 0), 0)

    const2 = lambda s, j: (0, 0)
    grid = (MAX_PATH, E + 1)
    cls, ent = pl.pallas_call(
        functools.partial(_moe_body, E),
        grid=grid,
        in_specs=[
            pl.BlockSpec((B, D), const2),
            pl.BlockSpec((D, HR), const2),
            pl.BlockSpec((1, HR), const2),
            pl.BlockSpec((D, W), const2),
            pl.BlockSpec((1, W), const2),
            pl.BlockSpec((1, D, HE), expert_map),
            pl.BlockSpec((1, 1, HE), expert_map),
            pl.BlockSpec((1, HE, D), expert_map),
            pl.BlockSpec((1, 1, D), expert_map),
            pl.BlockSpec((1, 1, D), expert_map),
            pl.BlockSpec((D, C), const2),
            pl.BlockSpec((1, C), const2),
            pl.BlockSpec(memory_space=pltpu.SMEM),
        ],
        out_specs=[
            pl.BlockSpec((B, C), const2),
            pl.BlockSpec((1, 1), const2),
        ],
        out_shape=[
            jax.ShapeDtypeStruct((B, C), jnp.float32),
            jax.ShapeDtypeStruct((1, 1), jnp.float32),
        ],
        scratch_shapes=[
            pltpu.VMEM((B, D), jnp.float32),
            pltpu.VMEM((B, D), jnp.float32),
            pltpu.VMEM((B, W), jnp.float32),
            pltpu.VMEM((B, 1), jnp.int32),
            pltpu.VMEM((B, 1), jnp.float32),
            pltpu.SMEM((1, 1), jnp.float32),
        ],
        compiler_params=pltpu.CompilerParams(
            dimension_semantics=("arbitrary", "arbitrary"),
            vmem_limit_bytes=63 * 1024 * 1024),
    )(x, Wr1, br1[None, :], Wr2p, br2p[None, :], We1, be1[:, None, :], We2,
      be2[:, None, :], tags[:, None, :], Wc, bc[None, :],
      jnp.ones((1, 1), jnp.float32))
    return cls, ent


def kernel(input_ids, emb, We1, be1, We2, be2, tags, Wr1, br1, Wr2, br2, Wc,
           bc):
    x = _embed_sc(input_ids, emb)
    cls, ent = _moe_tc(x, We1, be1, We2, be2, tags, Wr1, br1, Wr2, br2, Wc, bc)
    return cls, jnp.reshape(ent, ())


# speedup vs baseline: 1.2498x; 1.2498x over previous
import functools

import jax
import jax.numpy as jnp
from jax import lax
from jax.experimental import pallas as pl
from jax.experimental.pallas import tpu as pltpu
from jax.experimental.pallas import tpu_sc as plsc

MAX_PATH = 5
MAX_VISITS = 1
_NEG_INF = float("-inf")


def _emb_body(L, D, tok_per_w, ids_hbm, emb_hbm, out_hbm, idx_v, rows0_v,
              rows1_v, acc_v, sem0, sem1):
    nc = 2
    wid = lax.axis_index("s") * nc + lax.axis_index("c")
    pltpu.sync_copy(ids_hbm.at[wid], idx_v)

    inv = jnp.float32(1.0) / jnp.float32(L)

    def _fold_tree(vals):
        while len(vals) > 1:
            half = (len(vals) + 1) // 2
            vals = [vals[i] + vals[i + half] if i + half < len(vals)
                    else vals[i] for i in range(half)]
        return vals[0]

    def _accumulate(rows_v, t):
        def col_body(c4, _):
            for u in range(4):
                c = 4 * c4 + u
                acc = None
                for j in range(0, L, 8):
                    n = min(8, L - j)
                    vals = [rows_v[j + r, pl.ds(16 * c, 16)]
                            for r in range(n)]
                    tj = _fold_tree(vals)
                    acc = tj if acc is None else acc + tj
                acc_v[pl.ds(16 * c, 16)] = acc * inv
            return 0

        lax.fori_loop(0, D // 64, col_body, 0)
        pltpu.sync_copy(acc_v, out_hbm.at[wid * tok_per_w + t])

    def idx_at(t):
        return idx_v.at[t, pl.ds(0, L)]

    pltpu.async_copy(emb_hbm.at[idx_at(0)], rows0_v, sem0)

    def pair_body(i, _):
        t0 = 2 * i
        t1 = t0 + 1
        pltpu.async_copy(emb_hbm.at[idx_at(t1)], rows1_v, sem1)
        pltpu.make_async_copy(emb_hbm.at[idx_at(t0)], rows0_v, sem0).wait()
        _accumulate(rows0_v, t0)

        @pl.when(t0 + 2 < tok_per_w)
        def _next():
            pltpu.async_copy(emb_hbm.at[idx_at(t0 + 2)], rows0_v, sem0)

        pltpu.make_async_copy(emb_hbm.at[idx_at(t1)], rows1_v, sem1).wait()
        _accumulate(rows1_v, t1)
        return 0

    lax.fori_loop(0, tok_per_w // 2, pair_body, 0)


def _embed_sc(input_ids, emb):
    B, L = input_ids.shape
    V, D = emb.shape
    nw = 32
    tok_per_w = B // nw
    l_pad = (L + 7) // 8 * 8
    ids = jnp.pad(input_ids.astype(jnp.int32), ((0, 0), (0, l_pad - L)))
    ids = ids.reshape(nw, tok_per_w, l_pad)
    mesh = plsc.VectorSubcoreMesh(core_axis_name="c", subcore_axis_name="s")
    body = functools.partial(_emb_body, L, D, tok_per_w)
    return pl.kernel(
        body,
        out_type=jax.ShapeDtypeStruct((B, D), jnp.float32),
        mesh=mesh,
        scratch_types=[
            pltpu.VMEM((tok_per_w, l_pad), jnp.int32),
            pltpu.VMEM((L, D), jnp.float32),
            pltpu.VMEM((L, D), jnp.float32),
            pltpu.VMEM((D,), jnp.float32),
            pltpu.SemaphoreType.DMA,
            pltpu.SemaphoreType.DMA,
        ],
    )(ids, emb)


def _moe_body(E, x_in, Wr1, br1, Wr2, br2, We1, be1, We2, be2, tags, Wc, bc,
              one, cls_out, ent_out,
              x_s, out_s, visits_s, eidx_s, active_s, ent_s):
    s = pl.program_id(0)
    j = pl.program_id(1)
    B = x_s.shape[0]
    W = visits_s.shape[1]

    @pl.when((s == 0) & (j == 0))
    def _init():
        x_s[...] = x_in[...]
        visits_s[...] = jnp.zeros_like(visits_s)
        active_s[...] = jnp.ones_like(active_s)
        ent_s[0, 0] = 0.0

    @pl.when(j == 0)
    def _router():
        x = x_s[...]
        h = jnp.maximum(
            jnp.dot(x, Wr1[...], preferred_element_type=jnp.float32)
            + br1[...], 0.0)
        logits = (jnp.dot(h, Wr2[...], preferred_element_type=jnp.float32)
                  + br2[...])
        blocked = visits_s[...] >= MAX_VISITS
        logits = jnp.where(blocked, _NEG_INF, logits)
        m = jnp.max(logits, axis=1, keepdims=True)
        ex = jnp.exp(logits - m)
        probs = ex / jnp.sum(ex, axis=1, keepdims=True)
        ent = -jnp.sum(probs * jnp.log(probs + 1e-9), axis=1)
        af = active_s[...][:, 0]
        cnt = jnp.sum(af)
        step_ent = jnp.sum(ent * af) / jnp.maximum(cnt, 1.0)
        ent_s[0, 0] = ent_s[0, 0] + jnp.where(cnt > 0.0, step_ent, 0.0)
        sel = jnp.argmax(logits, axis=1).astype(jnp.int32)
        eidx = jnp.minimum(sel, E - 1)
        upd = (af > 0.0) & (sel != E)
        updf = upd.astype(jnp.float32)
        eidx_s[...] = eidx[:, None]
        active_s[...] = updf[:, None]
        col = lax.broadcasted_iota(jnp.int32, (B, W), 1)
        oh = ((col == eidx[:, None]) & upd[:, None]).astype(jnp.float32)
        visits_s[...] = visits_s[...] + oh
        out_s[...] = jnp.zeros_like(out_s)

    @pl.when(j > 0)
    def _expert():
        e = j - 1
        msk = (eidx_s[...] == e) & (active_s[...] > 0.0)

        @pl.when(jnp.any(msk))
        def _apply():
            D = x_s.shape[1]
            HE = We1.shape[2]
            s1 = one[0, 0]
            HH = HE // 2
            acc2 = None
            for half in range(2):
                hsl = slice(half * HH, (half + 1) * HH)
                acc1 = None
                for j in range(0, D, 128):
                    p = jnp.dot(x_s[:, j:j + 128], We1[0][j:j + 128, hsl],
                                preferred_element_type=jnp.float32) * s1
                    acc1 = p if acc1 is None else acc1 + p
                h1 = jnp.maximum(acc1 + be1[0][:, hsl], 0.0)
                for k in range(0, HH, 256):
                    ka = half * HH + k
                    pr = jnp.dot(h1[:, k:k + 128], We2[0][ka:ka + 128, :],
                                 preferred_element_type=jnp.float32) * s1
                    if k + 128 < HH:
                        pb = jnp.dot(h1[:, k + 128:k + 256],
                                     We2[0][ka + 128:ka + 256, :],
                                     preferred_element_type=jnp.float32) * s1
                        pr = pr + pb
                    acc2 = pr if acc2 is None else acc2 + pr
            o = acc2 + be2[0] + tags[0]
            out_s[...] = jnp.where(msk, o, out_s[...])

        @pl.when(j == E)
        def _finalize():
            xf = jnp.where(active_s[...] > 0.0, out_s[...], x_s[...])
            x_s[...] = xf

            @pl.when(s == MAX_PATH - 1)
            def _emit():
                cls_out[...] = (
                    jnp.dot(xf, Wc[...], preferred_element_type=jnp.float32)
                    + bc[...])
                ent_out[...] = jnp.full((1, 1), ent_s[0, 0], jnp.float32)


def _moe_tc(x, We1, be1, We2, be2, tags, Wr1, br1, Wr2, br2, Wc, bc):
    B, D = x.shape
    E, _, HE = We1.shape
    HR = Wr1.shape[1]
    C = Wc.shape[1]
    W = 16
    Wr2p = jnp.pad(Wr2, ((0, 0), (0, W - (E + 1))))
    br2p = jnp.pad(br2, (0, W - (E + 1)), constant_values=_NEG_INF)

    def expert_map(s, j):
        return (jnp.maximum(j - 1, 0), 0, 0)

    def expert_map2(s, j):
        return (jnp.maximum(j - 1, 0), 0)

    const2 = lambda s, j: (0, 0)
    grid = (MAX_PATH, E + 1)
    cls, ent = pl.pallas_call(
        functools.partial(_moe_body, E),
        grid=grid,
        in_specs=[
            pl.BlockSpec((B, D), const2),
            pl.BlockSpec((D, HR), const2),
            pl.BlockSpec((1, HR), const2),
            pl.BlockSpec((D, W), const2),
            pl.BlockSpec((1, W), const2),
            pl.BlockSpec((1, D, HE), expert_map),
            pl.BlockSpec((1, 1, HE), expert_map),
            pl.BlockSpec((1, HE, D), expert_map),
            pl.BlockSpec((1, 1, D), expert_map),
            pl.BlockSpec((1, 1, D), expert_map),
            pl.BlockSpec((D, C), const2),
            pl.BlockSpec((1, C), const2),
            pl.BlockSpec(memory_space=pltpu.SMEM),
        ],
        out_specs=[
            pl.BlockSpec((B, C), const2),
            pl.BlockSpec((1, 1), const2),
        ],
        out_shape=[
            jax.ShapeDtypeStruct((B, C), jnp.float32),
            jax.ShapeDtypeStruct((1, 1), jnp.float32),
        ],
        scratch_shapes=[
            pltpu.VMEM((B, D), jnp.float32),
            pltpu.VMEM((B, D), jnp.float32),
            pltpu.VMEM((B, W), jnp.float32),
            pltpu.VMEM((B, 1), jnp.int32),
            pltpu.VMEM((B, 1), jnp.float32),
            pltpu.SMEM((1, 1), jnp.float32),
        ],
        compiler_params=pltpu.CompilerParams(
            dimension_semantics=("arbitrary", "arbitrary"),
            vmem_limit_bytes=63 * 1024 * 1024),
    )(x, Wr1, br1[None, :], Wr2p, br2p[None, :], We1, be1[:, None, :], We2,
      be2[:, None, :], tags[:, None, :], Wc, bc[None, :],
      jnp.ones((1, 1), jnp.float32))
    return cls, ent


def kernel(input_ids, emb, We1, be1, We2, be2, tags, Wr1, br1, Wr2, br2, Wc,
           bc):
    x = _embed_sc(input_ids, emb)
    cls, ent = _moe_tc(x, We1, be1, We2, be2, tags, Wr1, br1, Wr2, br2, Wc, bc)
    return cls, jnp.reshape(ent, ())
